# Initial kernel scaffold; baseline (speedup 1.0000x reference)
import functools

import jax
import jax.numpy as jnp
from jax import lax
from jax.experimental import pallas as pl
from jax.experimental.pallas import tpu as pltpu
from jax.experimental.pallas import tpu_sc as plsc

N = 10000
D = 128
E = 320000
PRESERVE = 0.1

NC = 2
NS = 16
NW = NC * NS
EPW = E // NW
C = 80
K = EPW // C
ER = E // C
NP = 10240
ZW = NP // NS
RPT = N // NS
ZR = 125

_mesh = plsc.VectorSubcoreMesh(core_axis_name="c", subcore_axis_name="s")


@functools.partial(
    pl.kernel,
    out_type=jax.ShapeDtypeStruct((NC, NP), jnp.float32),
    mesh=_mesh,
    scratch_types=[
        pltpu.VMEM((K, C), jnp.int32),
        pltpu.VMEM((K, C), jnp.float32),
        pltpu.VMEM((ZW,), jnp.float32),
        pltpu.VMEM_SHARED((NP,), jnp.float32),
    ],
)
def _deg_kernel(col_hbm, ew_hbm, out_hbm, col_v, ew_v, zero_v, acc_sh):
    cid = lax.axis_index("c")
    sid = lax.axis_index("s")
    wid = sid * NC + cid

    def zbody(i, _):
        zero_v[pl.ds(i * 16, 16)] = jnp.zeros((16,), jnp.float32)
        return 0

    lax.fori_loop(0, ZW // 16, zbody, 0)
    pltpu.sync_copy(zero_v, acc_sh.at[pl.ds(sid * ZW, ZW)])
    plsc.subcore_barrier()

    pltpu.sync_copy(col_hbm.at[pl.ds(wid * K, K)], col_v)
    pltpu.sync_copy(ew_hbm.at[pl.ds(wid * K, K)], ew_v)

    def body(k, _):
        pltpu.sync_copy(ew_v.at[k], acc_sh.at[col_v.at[k]], add=True)
        return 0

    lax.fori_loop(0, K, body, 0)
    plsc.subcore_barrier()
    pltpu.sync_copy(acc_sh.at[pl.ds(sid * ZW, ZW)],
                    out_hbm.at[cid].at[pl.ds(sid * ZW, ZW)])


@functools.partial(
    pl.kernel,
    out_type=jax.ShapeDtypeStruct((NC, N, D), jnp.float32),
    mesh=_mesh,
    scratch_types=[
        pltpu.VMEM((K, C), jnp.int32),
        pltpu.VMEM((K, C), jnp.int32),
        pltpu.VMEM((K, C), jnp.float32),
        pltpu.VMEM((C, D), jnp.float32),
        pltpu.VMEM((ZR, D), jnp.float32),
        pltpu.VMEM_SHARED((N, D), jnp.float32),
        pltpu.SemaphoreType.DMA,
    ],
)
def _msg_kernel(hd_hbm, row_hbm, col_hbm, ew_hbm, out_hbm,
                row_v, col_v, ew_v, msg_v, zero_v, acc_sh, sem):
    cid = lax.axis_index("c")
    sid = lax.axis_index("s")
    wid = sid * NC + cid

    def zbody(i, _):
        for v in range(D // 16):
            zero_v[i, pl.ds(v * 16, 16)] = jnp.zeros((16,), jnp.float32)
        return 0

    lax.fori_loop(0, ZR, zbody, 0)
    for i in range(RPT // ZR):
        pltpu.sync_copy(zero_v, acc_sh.at[pl.ds(sid * RPT + i * ZR, ZR)])
    plsc.subcore_barrier()

    pltpu.sync_copy(row_hbm.at[pl.ds(wid * K, K)], row_v)
    pltpu.sync_copy(col_hbm.at[pl.ds(wid * K, K)], col_v)
    pltpu.sync_copy(ew_hbm.at[pl.ds(wid * K, K)], ew_v)

    def chunk(k, _):
        pltpu.async_copy(hd_hbm.at[row_v.at[k]], msg_v, sem).wait()

        def jbody(j, _):
            ewv = ew_v[k, pl.ds(j * 16, 16)]
            for l in range(16):
                b = jnp.take(ewv, jnp.full((16,), l, jnp.int32),
                             mode=lax.GatherScatterMode.PROMISE_IN_BOUNDS)
                for v in range(D // 16):
                    sl = pl.ds(v * 16, 16)
                    msg_v[j * 16 + l, sl] = msg_v[j * 16 + l, sl] * b
            return 0

        lax.fori_loop(0, C // 16, jbody, 0)
        pltpu.sync_copy(msg_v, acc_sh.at[col_v.at[k]], add=True)
        return 0

    lax.fori_loop(0, K, chunk, 0)
    plsc.subcore_barrier()
    for i in range(RPT // ZR):
        sl = pl.ds(sid * RPT + i * ZR, ZR)
        pltpu.sync_copy(acc_sh.at[sl], out_hbm.at[cid].at[sl])


RB = 2000
GRID = N // RB


def _tc_a_body(degp_ref, x_ref, w_ref, hd_ref, dis_ref):
    deg = degp_ref[0] + degp_ref[1] + 1.0
    dis = jnp.where(deg > 0.0, lax.rsqrt(deg), 0.0)
    dis = dis.reshape(RB, 1)
    dis_ref[...] = dis
    h = jnp.dot(x_ref[...], w_ref[...],
                preferred_element_type=jnp.float32,
                precision=lax.Precision.HIGHEST)
    hd_ref[...] = dis * h


def _tc_mid_body(accp_ref, hd_ref, dis_ref, b_ref, t_ref, w_ref,
                 tnew_ref, hdnew_ref):
    dis = dis_ref[...]
    agg = accp_ref[0] + accp_ref[1] + hd_ref[...]
    tnew = (1.0 - PRESERVE) * (dis * agg + b_ref[...]) + PRESERVE * t_ref[...]
    tnew_ref[...] = tnew
    h = jnp.dot(tnew, w_ref[...],
                preferred_element_type=jnp.float32,
                precision=lax.Precision.HIGHEST)
    hdnew_ref[...] = dis * h


def _tc_final_body(accp_ref, hd_ref, dis_ref, b_ref, t_ref, out_ref):
    dis = dis_ref[...]
    agg = accp_ref[0] + accp_ref[1] + hd_ref[...]
    out_ref[...] = ((1.0 - PRESERVE) * (dis * agg + b_ref[...])
                    + PRESERVE * t_ref[...])


_rowspec = pl.BlockSpec((RB, D), lambda i: (i, 0))
_disspec = pl.BlockSpec((RB, 1), lambda i: (i, 0))
_accspec = pl.BlockSpec((NC, RB, D), lambda i: (0, i, 0))
_wspec = pl.BlockSpec((D, D), lambda i: (0, 0))
_bspec = pl.BlockSpec((1, D), lambda i: (0, 0))

_tc_a = pl.pallas_call(
    _tc_a_body,
    grid=(GRID,),
    in_specs=[pl.BlockSpec((NC, RB), lambda i: (0, i)), _rowspec, _wspec],
    out_specs=[_rowspec, _disspec],
    out_shape=[jax.ShapeDtypeStruct((N, D), jnp.float32),
               jax.ShapeDtypeStruct((N, 1), jnp.float32)],
)

_tc_mid = pl.pallas_call(
    _tc_mid_body,
    grid=(GRID,),
    in_specs=[_accspec, _rowspec, _disspec, _bspec, _rowspec, _wspec],
    out_specs=[_rowspec, _rowspec],
    out_shape=[jax.ShapeDtypeStruct((N, D), jnp.float32),
               jax.ShapeDtypeStruct((N, D), jnp.float32)],
)

_tc_final = pl.pallas_call(
    _tc_final_body,
    grid=(GRID,),
    in_specs=[_accspec, _rowspec, _disspec, _bspec, _rowspec],
    out_specs=_rowspec,
    out_shape=jax.ShapeDtypeStruct((N, D), jnp.float32),
)


@jax.jit
def kernel(skill_embed, adj_list, edge_attr, W0, b0, W1, b1):
    row = adj_list[0].astype(jnp.int32).reshape(ER, C)
    col = adj_list[1].astype(jnp.int32).reshape(ER, C)
    ew = edge_attr.reshape(ER, C)

    deg_p = _deg_kernel(col, ew)
    hd0, dis = _tc_a(deg_p, skill_embed, W0)
    acc0 = _msg_kernel(hd0, row, col, ew)
    t1, hd1 = _tc_mid(acc0, hd0, dis, b0.reshape(1, D), skill_embed, W1)
    acc1 = _msg_kernel(hd1, row, col, ew)
    return _tc_final(acc1, hd1, dis, b1.reshape(1, D), t1)


# speedup vs baseline: 7.3120x; 7.3120x over previous
import functools

import jax
import jax.numpy as jnp
from jax import lax
from jax.experimental import pallas as pl
from jax.experimental.pallas import tpu as pltpu
from jax.experimental.pallas import tpu_sc as plsc

N = 10000
D = 128
E = 320000
PRESERVE = 0.1

NC = 2
NS = 16
NW = NC * NS
C = 128
K = 80
EPW = K * C
EP = NW * EPW
NP = 10240
ZW = NP // NS
RPT = NP // NS
NB = 2

_mesh = plsc.VectorSubcoreMesh(core_axis_name="c", subcore_axis_name="s")


@functools.partial(
    pl.kernel,
    out_type=jax.ShapeDtypeStruct((NC, NP), jnp.float32),
    mesh=_mesh,
    scratch_types=[
        pltpu.VMEM((K, C), jnp.int32),
        pltpu.VMEM((K, C), jnp.float32),
        pltpu.VMEM((ZW,), jnp.float32),
        pltpu.VMEM_SHARED((NP,), jnp.float32),
    ],
)
def _deg_kernel(col_hbm, ew_hbm, out_hbm, col_v, ew_v, zero_v, acc_sh):
    cid = lax.axis_index("c")
    sid = lax.axis_index("s")
    wid = sid * NC + cid

    def zbody(i, _):
        zero_v[pl.ds(i * 16, 16)] = jnp.zeros((16,), jnp.float32)
        return 0

    lax.fori_loop(0, ZW // 16, zbody, 0)
    pltpu.sync_copy(zero_v, acc_sh.at[pl.ds(sid * ZW, ZW)])
    plsc.subcore_barrier()

    pltpu.sync_copy(col_hbm.at[wid], col_v)
    pltpu.sync_copy(ew_hbm.at[wid], ew_v)

    def body(k, _):
        pltpu.sync_copy(ew_v.at[k], acc_sh.at[col_v.at[k]], add=True)
        return 0

    lax.fori_loop(0, K, body, 0)
    plsc.subcore_barrier()
    pltpu.sync_copy(acc_sh.at[pl.ds(sid * ZW, ZW)],
                    out_hbm.at[cid].at[pl.ds(sid * ZW, ZW)])


@functools.partial(
    pl.kernel,
    out_type=jax.ShapeDtypeStruct((NC, NP, D), jnp.float32),
    mesh=_mesh,
    scratch_types=[
        pltpu.VMEM((NB, C), jnp.int32),
        pltpu.VMEM((NB, C), jnp.int32),
        pltpu.VMEM((NB, C), jnp.float32),
        pltpu.VMEM((NB, C, D), jnp.float32),
        pltpu.VMEM_SHARED((NP, D), jnp.float32),
        pltpu.SemaphoreType.DMA((NB,)),
    ],
)
def _msg_kernel(hd_hbm, row_hbm, col_hbm, ew_hbm, out_hbm,
                row_v, col_v, ew_v, msg_v, acc_sh, sem):
    cid = lax.axis_index("c")
    sid = lax.axis_index("s")
    wid = sid * NC + cid

    def zbody(i, _):
        for v in range(D // 16):
            msg_v[0, i, pl.ds(v * 16, 16)] = jnp.zeros((16,), jnp.float32)
        return 0

    lax.fori_loop(0, C, zbody, 0)
    for i in range(RPT // C):
        pltpu.sync_copy(msg_v.at[0], acc_sh.at[pl.ds(sid * RPT + i * C, C)])
    plsc.subcore_barrier()

    def load_idx(k, buf):
        pltpu.sync_copy(row_hbm.at[wid].at[k], row_v.at[buf])
        pltpu.sync_copy(col_hbm.at[wid].at[k], col_v.at[buf])
        pltpu.sync_copy(ew_hbm.at[wid].at[k], ew_v.at[buf])

    load_idx(0, 0)
    pltpu.async_copy(hd_hbm.at[row_v.at[0]], msg_v.at[0], sem.at[0])

    def chunk(k, _):
        buf = lax.rem(k, NB)
        nbuf = lax.rem(k + 1, NB)

        @pl.when(k + 1 < K)
        def _prefetch():
            load_idx(k + 1, nbuf)
            pltpu.async_copy(hd_hbm.at[row_v.at[nbuf]], msg_v.at[nbuf],
                             sem.at[nbuf])

        pltpu.make_async_copy(hd_hbm.at[row_v.at[buf]], msg_v.at[buf],
                              sem.at[buf]).wait()

        def jbody(j, _):
            ewv = ew_v[buf, pl.ds(j * 16, 16)]
            for l in range(16):
                b = lax.gather(
                    ewv, jnp.full((16, 1), l, jnp.int32),
                    dimension_numbers=lax.GatherDimensionNumbers(
                        offset_dims=(), collapsed_slice_dims=(0,),
                        start_index_map=(0,)),
                    slice_sizes=(1,),
                    mode=lax.GatherScatterMode.PROMISE_IN_BOUNDS)
                for v in range(D // 16):
                    sl = pl.ds(v * 16, 16)
                    msg_v[buf, j * 16 + l, sl] = msg_v[buf, j * 16 + l, sl] * b
            return 0

        lax.fori_loop(0, C // 16, jbody, 0)
        pltpu.sync_copy(msg_v.at[buf], acc_sh.at[col_v.at[buf]], add=True)
        return 0

    lax.fori_loop(0, K, chunk, 0)
    plsc.subcore_barrier()
    for i in range(RPT // C):
        sl = pl.ds(sid * RPT + i * C, C)
        pltpu.sync_copy(acc_sh.at[sl], out_hbm.at[cid].at[sl])


RB = 2000
GRID = N // RB


def _tc_a_body(d0_ref, d1_ref, x_ref, w_ref, hd_ref, dis_ref):
    deg = d0_ref[...] + d1_ref[...] + 1.0
    dis = jnp.where(deg > 0.0, lax.rsqrt(deg), 0.0)
    dis_ref[...] = dis
    h = jnp.dot(x_ref[...], w_ref[...],
                preferred_element_type=jnp.float32,
                precision=lax.Precision.HIGHEST)
    hd_ref[...] = dis * h


def _tc_mid_body(accp_ref, hd_ref, dis_ref, b_ref, t_ref, w_ref,
                 tnew_ref, hdnew_ref):
    dis = dis_ref[...]
    agg = accp_ref[0] + accp_ref[1] + hd_ref[...]
    tnew = (1.0 - PRESERVE) * (dis * agg + b_ref[...]) + PRESERVE * t_ref[...]
    tnew_ref[...] = tnew
    h = jnp.dot(tnew, w_ref[...],
                preferred_element_type=jnp.float32,
                precision=lax.Precision.HIGHEST)
    hdnew_ref[...] = dis * h


def _tc_final_body(accp_ref, hd_ref, dis_ref, b_ref, t_ref, out_ref):
    dis = dis_ref[...]
    agg = accp_ref[0] + accp_ref[1] + hd_ref[...]
    out_ref[...] = ((1.0 - PRESERVE) * (dis * agg + b_ref[...])
                    + PRESERVE * t_ref[...])


_rowspec = pl.BlockSpec((RB, D), lambda i: (i, 0))
_disspec = pl.BlockSpec((RB, 1), lambda i: (i, 0))
_accspec = pl.BlockSpec((NC, RB, D), lambda i: (0, i, 0))
_wspec = pl.BlockSpec((D, D), lambda i: (0, 0))
_bspec = pl.BlockSpec((1, D), lambda i: (0, 0))

_tc_a = pl.pallas_call(
    _tc_a_body,
    grid=(GRID,),
    in_specs=[_disspec, _disspec, _rowspec, _wspec],
    out_specs=[_rowspec, _disspec],
    out_shape=[jax.ShapeDtypeStruct((N, D), jnp.float32),
               jax.ShapeDtypeStruct((N, 1), jnp.float32)],
)

_tc_mid = pl.pallas_call(
    _tc_mid_body,
    grid=(GRID,),
    in_specs=[_accspec, _rowspec, _disspec, _bspec, _rowspec, _wspec],
    out_specs=[_rowspec, _rowspec],
    out_shape=[jax.ShapeDtypeStruct((N, D), jnp.float32),
               jax.ShapeDtypeStruct((N, D), jnp.float32)],
)

_tc_final = pl.pallas_call(
    _tc_final_body,
    grid=(GRID,),
    in_specs=[_accspec, _rowspec, _disspec, _bspec, _rowspec],
    out_specs=_rowspec,
    out_shape=jax.ShapeDtypeStruct((N, D), jnp.float32),
)


@jax.jit
def kernel(skill_embed, adj_list, edge_attr, W0, b0, W1, b1):
    pad = EP - E
    zi = jnp.zeros((pad,), jnp.int32)
    row = jnp.concatenate([adj_list[0].astype(jnp.int32), zi]).reshape(NW, K, C)
    col = jnp.concatenate([adj_list[1].astype(jnp.int32), zi]).reshape(NW, K, C)
    ew = jnp.concatenate([edge_attr, jnp.zeros((pad,), jnp.float32)]
                         ).reshape(NW, K, C)

    deg_p = _deg_kernel(col, ew)
    d0 = deg_p[0, :N].reshape(N, 1)
    d1 = deg_p[1, :N].reshape(N, 1)
    hd0, dis = _tc_a(d0, d1, skill_embed, W0)
    acc0 = _msg_kernel(hd0, row, col, ew)
    t1, hd1 = _tc_mid(acc0, hd0, dis, b0.reshape(1, D), skill_embed, W1)
    acc1 = _msg_kernel(hd1, row, col, ew)
    return _tc_final(acc1, hd1, dis, b1.reshape(1, D), t1)


# speedup vs baseline: 9.7527x; 1.3338x over previous
import functools

import jax
import jax.numpy as jnp
from jax import lax
from jax.experimental import pallas as pl
from jax.experimental.pallas import tpu as pltpu
from jax.experimental.pallas import tpu_sc as plsc

N = 10000
D = 128
E = 320000
PRESERVE = 0.1

NC = 2
NS = 16
NW = NC * NS
C = 128
K0 = 96
K1 = 64
B1 = NS * K0
TCH = NS * (K0 + K1)
TOTC = 2592
G = 8
NB = 2
NP = 10240
ZW = NP // NS
RPT = NP // NS

_mesh = plsc.VectorSubcoreMesh(core_axis_name="c", subcore_axis_name="s")


@functools.partial(
    pl.kernel,
    out_type=jax.ShapeDtypeStruct((NC, NP), jnp.float32),
    mesh=_mesh,
    scratch_types=[
        pltpu.VMEM((K0, C), jnp.int32),
        pltpu.VMEM((K0, C), jnp.float32),
        pltpu.VMEM((ZW,), jnp.float32),
        pltpu.VMEM_SHARED((NP,), jnp.float32),
    ],
)
def _deg_kernel(col_hbm, ew_hbm, out_hbm, col_v, ew_v, zero_v, acc_sh):
    cid = lax.axis_index("c")
    sid = lax.axis_index("s")
    kw = jnp.where(cid == 0, K0, K1)
    base = pl.multiple_of(jnp.where(cid == 0, sid * K0, B1 + sid * K1), 8)

    def zbody(i, _):
        zero_v[pl.ds(i * 16, 16)] = jnp.zeros((16,), jnp.float32)
        return 0

    lax.fori_loop(0, ZW // 16, zbody, 0)
    pltpu.sync_copy(zero_v, acc_sh.at[pl.ds(sid * ZW, ZW)])
    plsc.subcore_barrier()

    pltpu.sync_copy(col_hbm.at[pl.ds(base, K0)], col_v)
    pltpu.sync_copy(ew_hbm.at[pl.ds(base, K0)], ew_v)

    def body(k, _):
        pltpu.sync_copy(ew_v.at[k], acc_sh.at[col_v.at[k]], add=True)
        return 0

    lax.fori_loop(0, kw, body, 0)
    plsc.subcore_barrier()
    pltpu.sync_copy(acc_sh.at[pl.ds(sid * ZW, ZW)],
                    out_hbm.at[cid].at[pl.ds(sid * ZW, ZW)])


@functools.partial(
    pl.kernel,
    out_type=jax.ShapeDtypeStruct((NC, NP, D), jnp.float32),
    mesh=_mesh,
    scratch_types=[
        pltpu.VMEM((NB, G, C), jnp.int32),
        pltpu.VMEM((NB, G, C), jnp.int32),
        pltpu.VMEM((NB, G, C), jnp.float32),
        pltpu.VMEM((NB, C, D), jnp.float32),
        pltpu.VMEM_SHARED((NP, D), jnp.float32),
        pltpu.SemaphoreType.DMA((NB,)),
        pltpu.SemaphoreType.DMA((NB,)),
        pltpu.SemaphoreType.DMA((NB,)),
    ],
)
def _msg_kernel(hd_hbm, row_hbm, col_hbm, ew_hbm, out_hbm,
                row_v, col_v, ew_v, msg_v, acc_sh, sem, ssem, isem):
    cid = lax.axis_index("c")
    sid = lax.axis_index("s")
    kw = jnp.where(cid == 0, K0, K1)
    gcnt = kw // G
    base = pl.multiple_of(jnp.where(cid == 0, sid * K0, B1 + sid * K1), 8)

    def zbody(i, _):
        for v in range(D // 16):
            msg_v[0, i, pl.ds(v * 16, 16)] = jnp.zeros((16,), jnp.float32)
        return 0

    lax.fori_loop(0, C, zbody, 0)
    for i in range(RPT // C):
        pltpu.sync_copy(msg_v.at[0], acc_sh.at[pl.ds(sid * RPT + i * C, C)])
    plsc.subcore_barrier()

    def load_group(g, gb):
        off = pl.multiple_of(base + g * G, 8)
        pltpu.async_copy(row_hbm.at[pl.ds(off, G)], row_v.at[gb], isem.at[gb])
        pltpu.async_copy(col_hbm.at[pl.ds(off, G)], col_v.at[gb], isem.at[gb])
        pltpu.async_copy(ew_hbm.at[pl.ds(off, G)], ew_v.at[gb], isem.at[gb])

    def wait_group(g, gb):
        off = pl.multiple_of(base + g * G, 8)
        pltpu.make_async_copy(row_hbm.at[pl.ds(off, G)], row_v.at[gb],
                              isem.at[gb]).wait()
        pltpu.make_async_copy(col_hbm.at[pl.ds(off, G)], col_v.at[gb],
                              isem.at[gb]).wait()
        pltpu.make_async_copy(ew_hbm.at[pl.ds(off, G)], ew_v.at[gb],
                              isem.at[gb]).wait()

    load_group(0, 0)
    wait_group(0, 0)
    load_group(1, 1)
    pltpu.async_copy(hd_hbm.at[row_v.at[0, 0]], msg_v.at[0], sem.at[0])

    def chunk(k, _):
        buf = lax.rem(k, NB)
        nbuf = lax.rem(k + 1, NB)
        j_in_g = lax.rem(k, G)
        gbuf = lax.rem(lax.div(k, G), NB)
        ngbuf = lax.rem(lax.div(k + 1, G), NB)
        nj = lax.rem(k + 1, G)

        @pl.when(k >= 1)
        def _drain_scatter():
            pj = lax.rem(k - 1, G)
            pgb = lax.rem(lax.div(k - 1, G), NB)
            pltpu.make_async_copy(msg_v.at[nbuf],
                                  acc_sh.at[col_v.at[pgb, pj]],
                                  ssem.at[nbuf]).wait()

        @pl.when((j_in_g == 0) & (k >= 1))
        def _issue_next_group():
            g_cur = lax.div(k, G)

            @pl.when(g_cur + 1 < gcnt)
            def _issue():
                load_group(g_cur + 1, lax.rem(g_cur + 1, NB))

        @pl.when((nj == 0) & (k + 1 < kw))
        def _wait_next_group():
            wait_group(lax.div(k + 1, G), ngbuf)

        @pl.when(k + 1 < kw)
        def _prefetch():
            pltpu.async_copy(hd_hbm.at[row_v.at[ngbuf, nj]], msg_v.at[nbuf],
                             sem.at[nbuf])

        pltpu.make_async_copy(hd_hbm.at[row_v.at[gbuf, j_in_g]],
                              msg_v.at[buf], sem.at[buf]).wait()

        mb = msg_v.at[buf]
        ewr = ew_v.at[gbuf, j_in_g]

        @plsc.parallel_loop(0, C // 16, unroll=2)
        def jbody(j):
            ewv = ewr[pl.ds(j * 16, 16)]
            for l in range(16):
                b = lax.gather(
                    ewv, jnp.full((16, 1), l, jnp.int32),
                    dimension_numbers=lax.GatherDimensionNumbers(
                        offset_dims=(), collapsed_slice_dims=(0,),
                        start_index_map=(0,)),
                    slice_sizes=(1,),
                    mode=lax.GatherScatterMode.PROMISE_IN_BOUNDS)
                e = j * 16 + l
                for v in range(D // 16):
                    sl = pl.ds(v * 16, 16)
                    mb[e, sl] = mb[e, sl] * b

        pltpu.async_copy(msg_v.at[buf], acc_sh.at[col_v.at[gbuf, j_in_g]],
                         ssem.at[buf], add=True)
        return 0

    lax.fori_loop(0, kw, chunk, 0)
    lb = lax.rem(kw - 1, NB)
    lgb = lax.rem(lax.div(kw - 1, G), NB)
    pltpu.make_async_copy(msg_v.at[lb],
                          acc_sh.at[col_v.at[lgb, lax.rem(kw - 1, G)]],
                          ssem.at[lb]).wait()
    plsc.subcore_barrier()
    for i in range(RPT // C):
        sl = pl.ds(sid * RPT + i * C, C)
        pltpu.sync_copy(acc_sh.at[sl], out_hbm.at[cid].at[sl])


RB = 2000
GRID = N // RB


def _tc_a_body(d0_ref, d1_ref, x_ref, w_ref, hd_ref, dis_ref):
    deg = d0_ref[...] + d1_ref[...] + 1.0
    dis = jnp.where(deg > 0.0, lax.rsqrt(deg), 0.0)
    dis_ref[...] = dis
    h = jnp.dot(x_ref[...], w_ref[...],
                preferred_element_type=jnp.float32,
                precision=lax.Precision.HIGHEST)
    hd_ref[...] = dis * h


def _tc_mid_body(accp_ref, hd_ref, dis_ref, b_ref, t_ref, w_ref,
                 tnew_ref, hdnew_ref):
    dis = dis_ref[...]
    agg = accp_ref[0] + accp_ref[1] + hd_ref[...]
    tnew = (1.0 - PRESERVE) * (dis * agg + b_ref[...]) + PRESERVE * t_ref[...]
    tnew_ref[...] = tnew
    h = jnp.dot(tnew, w_ref[...],
                preferred_element_type=jnp.float32,
                precision=lax.Precision.HIGHEST)
    hdnew_ref[...] = dis * h


def _tc_final_body(accp_ref, hd_ref, dis_ref, b_ref, t_ref, out_ref):
    dis = dis_ref[...]
    agg = accp_ref[0] + accp_ref[1] + hd_ref[...]
    out_ref[...] = ((1.0 - PRESERVE) * (dis * agg + b_ref[...])
                    + PRESERVE * t_ref[...])


_rowspec = pl.BlockSpec((RB, D), lambda i: (i, 0))
_disspec = pl.BlockSpec((RB, 1), lambda i: (i, 0))
_accspec = pl.BlockSpec((NC, RB, D), lambda i: (0, i, 0))
_wspec = pl.BlockSpec((D, D), lambda i: (0, 0))
_bspec = pl.BlockSpec((1, D), lambda i: (0, 0))

_tc_a = pl.pallas_call(
    _tc_a_body,
    grid=(GRID,),
    in_specs=[_disspec, _disspec, _rowspec, _wspec],
    out_specs=[_rowspec, _disspec],
    out_shape=[jax.ShapeDtypeStruct((N, D), jnp.float32),
               jax.ShapeDtypeStruct((N, 1), jnp.float32)],
)

_tc_mid = pl.pallas_call(
    _tc_mid_body,
    grid=(GRID,),
    in_specs=[_accspec, _rowspec, _disspec, _bspec, _rowspec, _wspec],
    out_specs=[_rowspec, _rowspec],
    out_shape=[jax.ShapeDtypeStruct((N, D), jnp.float32),
               jax.ShapeDtypeStruct((N, D), jnp.float32)],
)

_tc_final = pl.pallas_call(
    _tc_final_body,
    grid=(GRID,),
    in_specs=[_accspec, _rowspec, _disspec, _bspec, _rowspec],
    out_specs=_rowspec,
    out_shape=jax.ShapeDtypeStruct((N, D), jnp.float32),
)


@jax.jit
def kernel(skill_embed, adj_list, edge_attr, W0, b0, W1, b1):
    pad = TOTC * C - E
    zi = jnp.zeros((pad,), jnp.int32)
    row = jnp.concatenate([adj_list[0].astype(jnp.int32), zi]).reshape(TOTC, C)
    col = jnp.concatenate([adj_list[1].astype(jnp.int32), zi]).reshape(TOTC, C)
    ew = jnp.concatenate([edge_attr, jnp.zeros((pad,), jnp.float32)]
                         ).reshape(TOTC, C)

    deg_p = _deg_kernel(col, ew)
    d0 = deg_p[0, :N].reshape(N, 1)
    d1 = deg_p[1, :N].reshape(N, 1)
    hd0, dis = _tc_a(d0, d1, skill_embed, W0)
    acc0 = _msg_kernel(hd0, row, col, ew)
    t1, hd1 = _tc_mid(acc0, hd0, dis, b0.reshape(1, D), skill_embed, W1)
    acc1 = _msg_kernel(hd1, row, col, ew)
    return _tc_final(acc1, hd1, dis, b1.reshape(1, D), t1)


# speedup vs baseline: 9.8835x; 1.0134x over previous
import functools

import jax
import jax.numpy as jnp
from jax import lax
from jax.experimental import pallas as pl
from jax.experimental.pallas import tpu as pltpu
from jax.experimental.pallas import tpu_sc as plsc

N = 10000
D = 128
E = 320000
PRESERVE = 0.1

NC = 2
NS = 16
NW = NC * NS
C = 128
K0 = 104
K1 = 56
B1 = NS * K0
TCH = NS * (K0 + K1)
TOTC = 2592
G = 8
NB = 2
NP = 10240
ZW = NP // NS
RPT = NP // NS

_mesh = plsc.VectorSubcoreMesh(core_axis_name="c", subcore_axis_name="s")


@functools.partial(
    pl.kernel,
    out_type=jax.ShapeDtypeStruct((NC, NP), jnp.float32),
    mesh=_mesh,
    scratch_types=[
        pltpu.VMEM((K0, C), jnp.int32),
        pltpu.VMEM((K0, C), jnp.float32),
        pltpu.VMEM((ZW,), jnp.float32),
        pltpu.VMEM_SHARED((NP,), jnp.float32),
    ],
)
def _deg_kernel(col_hbm, ew_hbm, out_hbm, col_v, ew_v, zero_v, acc_sh):
    cid = lax.axis_index("c")
    sid = lax.axis_index("s")
    kw = jnp.where(cid == 0, K0, K1)
    base = pl.multiple_of(jnp.where(cid == 0, sid * K0, B1 + sid * K1), 8)

    def zbody(i, _):
        zero_v[pl.ds(i * 16, 16)] = jnp.zeros((16,), jnp.float32)
        return 0

    lax.fori_loop(0, ZW // 16, zbody, 0)
    pltpu.sync_copy(zero_v, acc_sh.at[pl.ds(sid * ZW, ZW)])
    plsc.subcore_barrier()

    pltpu.sync_copy(col_hbm.at[pl.ds(base, K0)], col_v)
    pltpu.sync_copy(ew_hbm.at[pl.ds(base, K0)], ew_v)

    def body(k, _):
        pltpu.sync_copy(ew_v.at[k], acc_sh.at[col_v.at[k]], add=True)
        return 0

    lax.fori_loop(0, kw, body, 0)
    plsc.subcore_barrier()
    pltpu.sync_copy(acc_sh.at[pl.ds(sid * ZW, ZW)],
                    out_hbm.at[cid].at[pl.ds(sid * ZW, ZW)])


@functools.partial(
    pl.kernel,
    out_type=jax.ShapeDtypeStruct((NC, NP, D), jnp.float32),
    mesh=_mesh,
    scratch_types=[
        pltpu.VMEM((NB, G, C), jnp.int32),
        pltpu.VMEM((NB, G, C), jnp.int32),
        pltpu.VMEM((NB, G, C), jnp.float32),
        pltpu.VMEM((NB, C, D), jnp.float32),
        pltpu.VMEM_SHARED((NP, D), jnp.float32),
        pltpu.SemaphoreType.DMA((NB,)),
        pltpu.SemaphoreType.DMA((NB,)),
        pltpu.SemaphoreType.DMA((NB,)),
    ],
)
def _msg_kernel(hd_hbm, row_hbm, col_hbm, ew_hbm, out_hbm,
                row_v, col_v, ew_v, msg_v, acc_sh, sem, ssem, isem):
    cid = lax.axis_index("c")
    sid = lax.axis_index("s")
    kw = jnp.where(cid == 0, K0, K1)
    gcnt = kw // G
    base = pl.multiple_of(jnp.where(cid == 0, sid * K0, B1 + sid * K1), 8)

    def zbody(i, _):
        for v in range(D // 16):
            msg_v[0, i, pl.ds(v * 16, 16)] = jnp.zeros((16,), jnp.float32)
        return 0

    lax.fori_loop(0, C, zbody, 0)
    for i in range(RPT // C):
        pltpu.sync_copy(msg_v.at[0], acc_sh.at[pl.ds(sid * RPT + i * C, C)])
    plsc.subcore_barrier()

    def load_group(g, gb):
        off = pl.multiple_of(base + g * G, 8)
        pltpu.async_copy(row_hbm.at[pl.ds(off, G)], row_v.at[gb], isem.at[gb])
        pltpu.async_copy(col_hbm.at[pl.ds(off, G)], col_v.at[gb], isem.at[gb])
        pltpu.async_copy(ew_hbm.at[pl.ds(off, G)], ew_v.at[gb], isem.at[gb])

    def wait_group(g, gb):
        off = pl.multiple_of(base + g * G, 8)
        pltpu.make_async_copy(row_hbm.at[pl.ds(off, G)], row_v.at[gb],
                              isem.at[gb]).wait()
        pltpu.make_async_copy(col_hbm.at[pl.ds(off, G)], col_v.at[gb],
                              isem.at[gb]).wait()
        pltpu.make_async_copy(ew_hbm.at[pl.ds(off, G)], ew_v.at[gb],
                              isem.at[gb]).wait()

    load_group(0, 0)
    wait_group(0, 0)
    load_group(1, 1)
    pltpu.async_copy(hd_hbm.at[row_v.at[0, 0]], msg_v.at[0], sem.at[0])

    def chunk(k, _):
        buf = lax.rem(k, NB)
        nbuf = lax.rem(k + 1, NB)
        j_in_g = lax.rem(k, G)
        gbuf = lax.rem(lax.div(k, G), NB)
        ngbuf = lax.rem(lax.div(k + 1, G), NB)
        nj = lax.rem(k + 1, G)

        @pl.when(k >= 1)
        def _drain_scatter():
            pj = lax.rem(k - 1, G)
            pgb = lax.rem(lax.div(k - 1, G), NB)
            pltpu.make_async_copy(msg_v.at[nbuf],
                                  acc_sh.at[col_v.at[pgb, pj]],
                                  ssem.at[nbuf]).wait()

        @pl.when((j_in_g == 0) & (k >= 1))
        def _issue_next_group():
            g_cur = lax.div(k, G)

            @pl.when(g_cur + 1 < gcnt)
            def _issue():
                load_group(g_cur + 1, lax.rem(g_cur + 1, NB))

        @pl.when((nj == 0) & (k + 1 < kw))
        def _wait_next_group():
            wait_group(lax.div(k + 1, G), ngbuf)

        @pl.when(k + 1 < kw)
        def _prefetch():
            pltpu.async_copy(hd_hbm.at[row_v.at[ngbuf, nj]], msg_v.at[nbuf],
                             sem.at[nbuf])

        pltpu.make_async_copy(hd_hbm.at[row_v.at[gbuf, j_in_g]],
                              msg_v.at[buf], sem.at[buf]).wait()

        mb = msg_v.at[buf]
        ewr = ew_v.at[gbuf, j_in_g]

        @plsc.parallel_loop(0, C // 16, unroll=2)
        def jbody(j):
            ewv = ewr[pl.ds(j * 16, 16)]
            for l in range(16):
                b = lax.gather(
                    ewv, jnp.full((16, 1), l, jnp.int32),
                    dimension_numbers=lax.GatherDimensionNumbers(
                        offset_dims=(), collapsed_slice_dims=(0,),
                        start_index_map=(0,)),
                    slice_sizes=(1,),
                    mode=lax.GatherScatterMode.PROMISE_IN_BOUNDS)
                e = j * 16 + l
                for v in range(D // 16):
                    sl = pl.ds(v * 16, 16)
                    mb[e, sl] = mb[e, sl] * b

        pltpu.async_copy(msg_v.at[buf], acc_sh.at[col_v.at[gbuf, j_in_g]],
                         ssem.at[buf], add=True)
        return 0

    lax.fori_loop(0, kw, chunk, 0)
    lb = lax.rem(kw - 1, NB)
    lgb = lax.rem(lax.div(kw - 1, G), NB)
    pltpu.make_async_copy(msg_v.at[lb],
                          acc_sh.at[col_v.at[lgb, lax.rem(kw - 1, G)]],
                          ssem.at[lb]).wait()
    plsc.subcore_barrier()
    for i in range(RPT // C):
        sl = pl.ds(sid * RPT + i * C, C)
        pltpu.sync_copy(acc_sh.at[sl], out_hbm.at[cid].at[sl])


RB = 2000
GRID = N // RB


def _tc_a_body(d0_ref, d1_ref, x_ref, w_ref, hd_ref, dis_ref):
    deg = d0_ref[...] + d1_ref[...] + 1.0
    dis = jnp.where(deg > 0.0, lax.rsqrt(deg), 0.0)
    dis_ref[...] = dis
    h = jnp.dot(x_ref[...], w_ref[...],
                preferred_element_type=jnp.float32,
                precision=lax.Precision.HIGHEST)
    hd_ref[...] = dis * h


def _tc_mid_body(accp_ref, hd_ref, dis_ref, b_ref, t_ref, w_ref,
                 tnew_ref, hdnew_ref):
    dis = dis_ref[...]
    agg = accp_ref[0] + accp_ref[1] + hd_ref[...]
    tnew = (1.0 - PRESERVE) * (dis * agg + b_ref[...]) + PRESERVE * t_ref[...]
    tnew_ref[...] = tnew
    h = jnp.dot(tnew, w_ref[...],
                preferred_element_type=jnp.float32,
                precision=lax.Precision.HIGHEST)
    hdnew_ref[...] = dis * h


def _tc_final_body(accp_ref, hd_ref, dis_ref, b_ref, t_ref, out_ref):
    dis = dis_ref[...]
    agg = accp_ref[0] + accp_ref[1] + hd_ref[...]
    out_ref[...] = ((1.0 - PRESERVE) * (dis * agg + b_ref[...])
                    + PRESERVE * t_ref[...])


_rowspec = pl.BlockSpec((RB, D), lambda i: (i, 0))
_disspec = pl.BlockSpec((RB, 1), lambda i: (i, 0))
_accspec = pl.BlockSpec((NC, RB, D), lambda i: (0, i, 0))
_wspec = pl.BlockSpec((D, D), lambda i: (0, 0))
_bspec = pl.BlockSpec((1, D), lambda i: (0, 0))

_tc_a = pl.pallas_call(
    _tc_a_body,
    grid=(GRID,),
    in_specs=[_disspec, _disspec, _rowspec, _wspec],
    out_specs=[_rowspec, _disspec],
    out_shape=[jax.ShapeDtypeStruct((N, D), jnp.float32),
               jax.ShapeDtypeStruct((N, 1), jnp.float32)],
)

_tc_mid = pl.pallas_call(
    _tc_mid_body,
    grid=(GRID,),
    in_specs=[_accspec, _rowspec, _disspec, _bspec, _rowspec, _wspec],
    out_specs=[_rowspec, _rowspec],
    out_shape=[jax.ShapeDtypeStruct((N, D), jnp.float32),
               jax.ShapeDtypeStruct((N, D), jnp.float32)],
)

_tc_final = pl.pallas_call(
    _tc_final_body,
    grid=(GRID,),
    in_specs=[_accspec, _rowspec, _disspec, _bspec, _rowspec],
    out_specs=_rowspec,
    out_shape=jax.ShapeDtypeStruct((N, D), jnp.float32),
)


@jax.jit
def kernel(skill_embed, adj_list, edge_attr, W0, b0, W1, b1):
    pad = TOTC * C - E
    zi = jnp.zeros((pad,), jnp.int32)
    row = jnp.concatenate([adj_list[0].astype(jnp.int32), zi]).reshape(TOTC, C)
    col = jnp.concatenate([adj_list[1].astype(jnp.int32), zi]).reshape(TOTC, C)
    ew = jnp.concatenate([edge_attr, jnp.zeros((pad,), jnp.float32)]
                         ).reshape(TOTC, C)

    deg_p = _deg_kernel(col, ew)
    d0 = deg_p[0, :N].reshape(N, 1)
    d1 = deg_p[1, :N].reshape(N, 1)
    hd0, dis = _tc_a(d0, d1, skill_embed, W0)
    acc0 = _msg_kernel(hd0, row, col, ew)
    t1, hd1 = _tc_mid(acc0, hd0, dis, b0.reshape(1, D), skill_embed, W1)
    acc1 = _msg_kernel(hd1, row, col, ew)
    return _tc_final(acc1, hd1, dis, b1.reshape(1, D), t1)


# speedup vs baseline: 9.9320x; 1.0049x over previous
import functools

import jax
import jax.numpy as jnp
from jax import lax
from jax.experimental import pallas as pl
from jax.experimental.pallas import tpu as pltpu
from jax.experimental.pallas import tpu_sc as plsc

N = 10000
D = 128
E = 320000
PRESERVE = 0.1

NC = 2
NS = 16
NW = NC * NS
C = 128
K0 = 112
K1 = 48
B1 = NS * K0
TCH = NS * (K0 + K1)
TOTC = 2592
G = 8
NB = 2
NP = 10240
ZW = NP // NS
RPT = NP // NS

_mesh = plsc.VectorSubcoreMesh(core_axis_name="c", subcore_axis_name="s")


@functools.partial(
    pl.kernel,
    out_type=jax.ShapeDtypeStruct((NC, NP), jnp.float32),
    mesh=_mesh,
    scratch_types=[
        pltpu.VMEM((K0, C), jnp.int32),
        pltpu.VMEM((K0, C), jnp.float32),
        pltpu.VMEM((ZW,), jnp.float32),
        pltpu.VMEM_SHARED((NP,), jnp.float32),
    ],
)
def _deg_kernel(col_hbm, ew_hbm, out_hbm, col_v, ew_v, zero_v, acc_sh):
    cid = lax.axis_index("c")
    sid = lax.axis_index("s")
    kw = jnp.where(cid == 0, K0, K1)
    base = pl.multiple_of(jnp.where(cid == 0, sid * K0, B1 + sid * K1), 8)

    def zbody(i, _):
        zero_v[pl.ds(i * 16, 16)] = jnp.zeros((16,), jnp.float32)
        return 0

    lax.fori_loop(0, ZW // 16, zbody, 0)
    pltpu.sync_copy(zero_v, acc_sh.at[pl.ds(sid * ZW, ZW)])
    plsc.subcore_barrier()

    pltpu.sync_copy(col_hbm.at[pl.ds(base, K0)], col_v)
    pltpu.sync_copy(ew_hbm.at[pl.ds(base, K0)], ew_v)

    def body(k, _):
        pltpu.sync_copy(ew_v.at[k], acc_sh.at[col_v.at[k]], add=True)
        return 0

    lax.fori_loop(0, kw, body, 0)
    plsc.subcore_barrier()
    pltpu.sync_copy(acc_sh.at[pl.ds(sid * ZW, ZW)],
                    out_hbm.at[cid].at[pl.ds(sid * ZW, ZW)])


@functools.partial(
    pl.kernel,
    out_type=jax.ShapeDtypeStruct((NC, NP, D), jnp.float32),
    mesh=_mesh,
    scratch_types=[
        pltpu.VMEM((NB, G, C), jnp.int32),
        pltpu.VMEM((NB, G, C), jnp.int32),
        pltpu.VMEM((NB, G, C), jnp.float32),
        pltpu.VMEM((NB, C, D), jnp.float32),
        pltpu.VMEM_SHARED((NP, D), jnp.float32),
        pltpu.SemaphoreType.DMA((NB,)),
        pltpu.SemaphoreType.DMA((NB,)),
        pltpu.SemaphoreType.DMA((NB,)),
    ],
)
def _msg_kernel(hd_hbm, row_hbm, col_hbm, ew_hbm, out_hbm,
                row_v, col_v, ew_v, msg_v, acc_sh, sem, ssem, isem):
    cid = lax.axis_index("c")
    sid = lax.axis_index("s")
    kw = jnp.where(cid == 0, K0, K1)
    gcnt = kw // G
    base = pl.multiple_of(jnp.where(cid == 0, sid * K0, B1 + sid * K1), 8)

    def zbody(i, _):
        for v in range(D // 16):
            msg_v[0, i, pl.ds(v * 16, 16)] = jnp.zeros((16,), jnp.float32)
        return 0

    lax.fori_loop(0, C, zbody, 0)
    for i in range(RPT // C):
        pltpu.sync_copy(msg_v.at[0], acc_sh.at[pl.ds(sid * RPT + i * C, C)])
    plsc.subcore_barrier()

    def load_group(g, gb):
        off = pl.multiple_of(base + g * G, 8)
        pltpu.async_copy(row_hbm.at[pl.ds(off, G)], row_v.at[gb], isem.at[gb])
        pltpu.async_copy(col_hbm.at[pl.ds(off, G)], col_v.at[gb], isem.at[gb])
        pltpu.async_copy(ew_hbm.at[pl.ds(off, G)], ew_v.at[gb], isem.at[gb])

    def wait_group(g, gb):
        off = pl.multiple_of(base + g * G, 8)
        pltpu.make_async_copy(row_hbm.at[pl.ds(off, G)], row_v.at[gb],
                              isem.at[gb]).wait()
        pltpu.make_async_copy(col_hbm.at[pl.ds(off, G)], col_v.at[gb],
                              isem.at[gb]).wait()
        pltpu.make_async_copy(ew_hbm.at[pl.ds(off, G)], ew_v.at[gb],
                              isem.at[gb]).wait()

    load_group(0, 0)
    wait_group(0, 0)
    load_group(1, 1)
    pltpu.async_copy(hd_hbm.at[row_v.at[0, 0]], msg_v.at[0], sem.at[0])

    def chunk(k, _):
        buf = lax.rem(k, NB)
        nbuf = lax.rem(k + 1, NB)
        j_in_g = lax.rem(k, G)
        gbuf = lax.rem(lax.div(k, G), NB)
        ngbuf = lax.rem(lax.div(k + 1, G), NB)
        nj = lax.rem(k + 1, G)

        @pl.when(k >= 1)
        def _drain_scatter():
            pj = lax.rem(k - 1, G)
            pgb = lax.rem(lax.div(k - 1, G), NB)
            pltpu.make_async_copy(msg_v.at[nbuf],
                                  acc_sh.at[col_v.at[pgb, pj]],
                                  ssem.at[nbuf]).wait()

        @pl.when((j_in_g == 0) & (k >= 1))
        def _issue_next_group():
            g_cur = lax.div(k, G)

            @pl.when(g_cur + 1 < gcnt)
            def _issue():
                load_group(g_cur + 1, lax.rem(g_cur + 1, NB))

        @pl.when((nj == 0) & (k + 1 < kw))
        def _wait_next_group():
            wait_group(lax.div(k + 1, G), ngbuf)

        @pl.when(k + 1 < kw)
        def _prefetch():
            pltpu.async_copy(hd_hbm.at[row_v.at[ngbuf, nj]], msg_v.at[nbuf],
                             sem.at[nbuf])

        pltpu.make_async_copy(hd_hbm.at[row_v.at[gbuf, j_in_g]],
                              msg_v.at[buf], sem.at[buf]).wait()

        mb = msg_v.at[buf]
        ewr = ew_v.at[gbuf, j_in_g]

        @plsc.parallel_loop(0, C // 16, unroll=2)
        def jbody(j):
            ewv = ewr[pl.ds(j * 16, 16)]
            for l in range(16):
                b = lax.gather(
                    ewv, jnp.full((16, 1), l, jnp.int32),
                    dimension_numbers=lax.GatherDimensionNumbers(
                        offset_dims=(), collapsed_slice_dims=(0,),
                        start_index_map=(0,)),
                    slice_sizes=(1,),
                    mode=lax.GatherScatterMode.PROMISE_IN_BOUNDS)
                e = j * 16 + l
                for v in range(D // 16):
                    sl = pl.ds(v * 16, 16)
                    mb[e, sl] = mb[e, sl] * b

        pltpu.async_copy(msg_v.at[buf], acc_sh.at[col_v.at[gbuf, j_in_g]],
                         ssem.at[buf], add=True)
        return 0

    lax.fori_loop(0, kw, chunk, 0)
    lb = lax.rem(kw - 1, NB)
    lgb = lax.rem(lax.div(kw - 1, G), NB)
    pltpu.make_async_copy(msg_v.at[lb],
                          acc_sh.at[col_v.at[lgb, lax.rem(kw - 1, G)]],
                          ssem.at[lb]).wait()
    plsc.subcore_barrier()
    for i in range(RPT // C):
        sl = pl.ds(sid * RPT + i * C, C)
        pltpu.sync_copy(acc_sh.at[sl], out_hbm.at[cid].at[sl])


RB = 2000
GRID = N // RB


def _tc_a_body(d0_ref, d1_ref, x_ref, w_ref, hd_ref, dis_ref):
    deg = d0_ref[...] + d1_ref[...] + 1.0
    dis = jnp.where(deg > 0.0, lax.rsqrt(deg), 0.0)
    dis_ref[...] = dis
    h = jnp.dot(x_ref[...], w_ref[...],
                preferred_element_type=jnp.float32,
                precision=lax.Precision.HIGHEST)
    hd_ref[...] = dis * h


def _tc_mid_body(accp_ref, hd_ref, dis_ref, b_ref, t_ref, w_ref,
                 tnew_ref, hdnew_ref):
    dis = dis_ref[...]
    agg = accp_ref[0] + accp_ref[1] + hd_ref[...]
    tnew = (1.0 - PRESERVE) * (dis * agg + b_ref[...]) + PRESERVE * t_ref[...]
    tnew_ref[...] = tnew
    h = jnp.dot(tnew, w_ref[...],
                preferred_element_type=jnp.float32,
                precision=lax.Precision.HIGHEST)
    hdnew_ref[...] = dis * h


def _tc_final_body(accp_ref, hd_ref, dis_ref, b_ref, t_ref, out_ref):
    dis = dis_ref[...]
    agg = accp_ref[0] + accp_ref[1] + hd_ref[...]
    out_ref[...] = ((1.0 - PRESERVE) * (dis * agg + b_ref[...])
                    + PRESERVE * t_ref[...])


_rowspec = pl.BlockSpec((RB, D), lambda i: (i, 0))
_disspec = pl.BlockSpec((RB, 1), lambda i: (i, 0))
_accspec = pl.BlockSpec((NC, RB, D), lambda i: (0, i, 0))
_wspec = pl.BlockSpec((D, D), lambda i: (0, 0))
_bspec = pl.BlockSpec((1, D), lambda i: (0, 0))

_tc_a = pl.pallas_call(
    _tc_a_body,
    grid=(GRID,),
    in_specs=[_disspec, _disspec, _rowspec, _wspec],
    out_specs=[_rowspec, _disspec],
    out_shape=[jax.ShapeDtypeStruct((N, D), jnp.float32),
               jax.ShapeDtypeStruct((N, 1), jnp.float32)],
)

_tc_mid = pl.pallas_call(
    _tc_mid_body,
    grid=(GRID,),
    in_specs=[_accspec, _rowspec, _disspec, _bspec, _rowspec, _wspec],
    out_specs=[_rowspec, _rowspec],
    out_shape=[jax.ShapeDtypeStruct((N, D), jnp.float32),
               jax.ShapeDtypeStruct((N, D), jnp.float32)],
)

_tc_final = pl.pallas_call(
    _tc_final_body,
    grid=(GRID,),
    in_specs=[_accspec, _rowspec, _disspec, _bspec, _rowspec],
    out_specs=_rowspec,
    out_shape=jax.ShapeDtypeStruct((N, D), jnp.float32),
)


@jax.jit
def kernel(skill_embed, adj_list, edge_attr, W0, b0, W1, b1):
    pad = TOTC * C - E
    zi = jnp.zeros((pad,), jnp.int32)
    row = jnp.concatenate([adj_list[0].astype(jnp.int32), zi]).reshape(TOTC, C)
    col = jnp.concatenate([adj_list[1].astype(jnp.int32), zi]).reshape(TOTC, C)
    ew = jnp.concatenate([edge_attr, jnp.zeros((pad,), jnp.float32)]
                         ).reshape(TOTC, C)

    deg_p = _deg_kernel(col, ew)
    d0 = deg_p[0, :N].reshape(N, 1)
    d1 = deg_p[1, :N].reshape(N, 1)
    hd0, dis = _tc_a(d0, d1, skill_embed, W0)
    acc0 = _msg_kernel(hd0, row, col, ew)
    t1, hd1 = _tc_mid(acc0, hd0, dis, b0.reshape(1, D), skill_embed, W1)
    acc1 = _msg_kernel(hd1, row, col, ew)
    return _tc_final(acc1, hd1, dis, b1.reshape(1, D), t1)
